# Initial kernel scaffold; baseline (speedup 1.0000x reference)
import functools

import jax
import jax.numpy as jnp
from jax import lax
from jax.experimental import pallas as pl
from jax.experimental.pallas import tpu as pltpu
from jax.experimental.pallas import tpu_sc as plsc

B = 16384
V = 100000
D = 64
PD = 64
NNEG = 5

NC = 2
NS = 16
NW = NC * NS
BPW = B // NW


def _sc_gather_kernel(pu0, pu1, pv, negf, u_emb, v_emb,
                      w1_out, w2_out, vv_out, nv_out,
                      idx_v, rows_v, nidx_v, nrows_v, sem):
    wid = lax.axis_index("s") * NC + lax.axis_index("c")
    base = wid * BPW

    pltpu.sync_copy(pu0.at[pl.ds(base, BPW)], idx_v)
    pltpu.async_copy(u_emb.at[idx_v], rows_v, sem).wait()
    pltpu.sync_copy(rows_v, w1_out.at[pl.ds(base, BPW)])

    pltpu.sync_copy(pu1.at[pl.ds(base, BPW)], idx_v)
    pltpu.async_copy(u_emb.at[idx_v], rows_v, sem).wait()
    pltpu.sync_copy(rows_v, w2_out.at[pl.ds(base, BPW)])

    pltpu.sync_copy(pv.at[pl.ds(base, BPW)], idx_v)
    pltpu.async_copy(v_emb.at[idx_v], rows_v, sem).wait()
    pltpu.sync_copy(rows_v, vv_out.at[pl.ds(base, BPW)])

    nbase = base * NNEG
    pltpu.sync_copy(negf.at[pl.ds(nbase, NNEG * BPW)], nidx_v)
    pltpu.async_copy(v_emb.at[nidx_v], nrows_v, sem).wait()
    pltpu.sync_copy(nrows_v, nv_out.at[pl.ds(nbase, NNEG * BPW)])


@jax.jit
def _sc_gather(pu0, pu1, pv, negf, u_emb, v_emb):
    mesh = plsc.VectorSubcoreMesh(core_axis_name="c", subcore_axis_name="s")
    f = pl.kernel(
        _sc_gather_kernel,
        out_type=[
            jax.ShapeDtypeStruct((B, D), jnp.float32),
            jax.ShapeDtypeStruct((B, D), jnp.float32),
            jax.ShapeDtypeStruct((B, PD), jnp.float32),
            jax.ShapeDtypeStruct((B * NNEG, PD), jnp.float32),
        ],
        mesh=mesh,
        scratch_types=[
            pltpu.VMEM((BPW,), jnp.int32),
            pltpu.VMEM((BPW, D), jnp.float32),
            pltpu.VMEM((NNEG * BPW,), jnp.int32),
            pltpu.VMEM((NNEG * BPW, PD), jnp.float32),
            pltpu.SemaphoreType.DMA,
        ],
    )
    return f(pu0, pu1, pv, negf, u_emb, v_emb)


def _log_sigmoid(x):
    return jnp.minimum(x, 0.0) - jnp.log1p(jnp.exp(-jnp.abs(x)))


BS = 2048


def _tc_loss_kernel(w1_ref, w2_ref, vv_ref, nv_ref, W_ref, b_ref, out_ref):
    i = pl.program_id(0)

    w1 = w1_ref[...]
    w2 = w2_ref[...]
    rel = lax.dot_general(w1, W_ref[:, :D], (((1,), (1,)), ((), ())),
                          preferred_element_type=jnp.float32)
    rel += lax.dot_general(w2, W_ref[:, D:], (((1,), (1,)), ((), ())),
                           preferred_element_type=jnp.float32)
    rel += b_ref[...]

    score = jnp.sum(rel * vv_ref[...], axis=1)
    pos = jnp.sum(_log_sigmoid(score))

    ns = jnp.sum(nv_ref[...] * rel[:, None, :], axis=2)
    neg = jnp.sum(_log_sigmoid(-ns))

    @pl.when(i == 0)
    def _():
        out_ref[0, 0] = 0.0

    out_ref[0, 0] += pos + neg


@jax.jit
def _tc_loss(w1, w2, vv, nv3, W, b2):
    grid = (B // BS,)
    out = pl.pallas_call(
        _tc_loss_kernel,
        grid=grid,
        in_specs=[
            pl.BlockSpec((BS, D), lambda i: (i, 0)),
            pl.BlockSpec((BS, D), lambda i: (i, 0)),
            pl.BlockSpec((BS, PD), lambda i: (i, 0)),
            pl.BlockSpec((BS, NNEG, PD), lambda i: (i, 0, 0)),
            pl.BlockSpec((PD, 2 * D), lambda i: (0, 0)),
            pl.BlockSpec((1, PD), lambda i: (0, 0)),
        ],
        out_specs=pl.BlockSpec((1, 1), lambda i: (0, 0)),
        out_shape=jax.ShapeDtypeStruct((1, 1), jnp.float32),
    )(w1, w2, vv, nv3, W, b2)
    return out


def kernel(pos_u, pos_v, neg_v, u_emb, W, b, v_emb):
    pu0 = pos_u[:, 0].astype(jnp.int32)
    pu1 = pos_u[:, 1].astype(jnp.int32)
    pv = pos_v.astype(jnp.int32)
    negf = neg_v.reshape(-1).astype(jnp.int32)

    w1, w2, vv, nv = _sc_gather(pu0, pu1, pv, negf, u_emb, v_emb)
    nv3 = nv.reshape(B, NNEG, PD)
    b2 = b.reshape(1, PD)

    out = _tc_loss(w1, w2, vv, nv3, W, b2)
    return -out[0, 0]


# speedup vs baseline: 1.2875x; 1.2875x over previous
import functools

import jax
import jax.numpy as jnp
from jax import lax
from jax.experimental import pallas as pl
from jax.experimental.pallas import tpu as pltpu
from jax.experimental.pallas import tpu_sc as plsc

B = 16384
V = 100000
D = 64
PD = 64
NNEG = 5

NC = 2
NS = 16
NW = NC * NS
BPW = B // NW


def _sc_gather_kernel(pu0, pu1, pv, negf, u_emb, v_emb,
                      w1_out, w2_out, vv_out, nv_out,
                      idx_v, rows_v, sem):
    wid = lax.axis_index("s") * NC + lax.axis_index("c")
    base = wid * BPW

    pltpu.sync_copy(pu0.at[pl.ds(base, BPW)], idx_v)
    pltpu.async_copy(u_emb.at[idx_v], rows_v, sem).wait()
    pltpu.sync_copy(rows_v, w1_out.at[pl.ds(base, BPW)])

    pltpu.sync_copy(pu1.at[pl.ds(base, BPW)], idx_v)
    pltpu.async_copy(u_emb.at[idx_v], rows_v, sem).wait()
    pltpu.sync_copy(rows_v, w2_out.at[pl.ds(base, BPW)])

    pltpu.sync_copy(pv.at[pl.ds(base, BPW)], idx_v)
    pltpu.async_copy(v_emb.at[idx_v], rows_v, sem).wait()
    pltpu.sync_copy(rows_v, vv_out.at[pl.ds(base, BPW)])

    for j in range(NNEG):
        nb = base * NNEG + j * BPW
        pltpu.sync_copy(negf.at[pl.ds(nb, BPW)], idx_v)
        pltpu.async_copy(v_emb.at[idx_v], rows_v, sem).wait()
        pltpu.sync_copy(rows_v, nv_out.at[pl.ds(nb, BPW)])


@jax.jit
def _sc_gather(pu0, pu1, pv, negf, u_emb, v_emb):
    mesh = plsc.VectorSubcoreMesh(core_axis_name="c", subcore_axis_name="s")
    f = pl.kernel(
        _sc_gather_kernel,
        out_type=[
            jax.ShapeDtypeStruct((B, D), jnp.float32),
            jax.ShapeDtypeStruct((B, D), jnp.float32),
            jax.ShapeDtypeStruct((B, PD), jnp.float32),
            jax.ShapeDtypeStruct((B * NNEG, PD), jnp.float32),
        ],
        mesh=mesh,
        scratch_types=[
            pltpu.VMEM((BPW,), jnp.int32),
            pltpu.VMEM((BPW, D), jnp.float32),
            pltpu.SemaphoreType.DMA,
        ],
        compiler_params=pltpu.CompilerParams(use_tc_tiling_on_sc=False),
    )
    return f(pu0, pu1, pv, negf, u_emb, v_emb)


def _log_sigmoid(x):
    return jnp.minimum(x, 0.0) - jnp.log1p(jnp.exp(-jnp.abs(x)))


BS = 2048


def _tc_loss_kernel(w1_ref, w2_ref, vv_ref, nv_ref, W_ref, b_ref, out_ref):
    i = pl.program_id(0)

    w1 = w1_ref[...]
    w2 = w2_ref[...]
    rel = lax.dot_general(w1, W_ref[:, :D], (((1,), (1,)), ((), ())),
                          preferred_element_type=jnp.float32)
    rel += lax.dot_general(w2, W_ref[:, D:], (((1,), (1,)), ((), ())),
                           preferred_element_type=jnp.float32)
    rel += b_ref[...]

    score = jnp.sum(rel * vv_ref[...], axis=1)
    pos = jnp.sum(_log_sigmoid(score))

    ns = jnp.sum(nv_ref[...] * rel[:, None, :], axis=2)
    neg = jnp.sum(_log_sigmoid(-ns))

    @pl.when(i == 0)
    def _():
        out_ref[...] = jnp.zeros((1, 1), jnp.float32)

    out_ref[...] += jnp.broadcast_to(pos + neg, (1, 1))


@jax.jit
def _tc_loss(w1, w2, vv, nv3, W, b2):
    grid = (B // BS,)
    out = pl.pallas_call(
        _tc_loss_kernel,
        grid=grid,
        in_specs=[
            pl.BlockSpec((BS, D), lambda i: (i, 0)),
            pl.BlockSpec((BS, D), lambda i: (i, 0)),
            pl.BlockSpec((BS, PD), lambda i: (i, 0)),
            pl.BlockSpec((BS, NNEG, PD), lambda i: (i, 0, 0)),
            pl.BlockSpec((PD, 2 * D), lambda i: (0, 0)),
            pl.BlockSpec((1, PD), lambda i: (0, 0)),
        ],
        out_specs=pl.BlockSpec((1, 1), lambda i: (0, 0)),
        out_shape=jax.ShapeDtypeStruct((1, 1), jnp.float32),
    )(w1, w2, vv, nv3, W, b2)
    return out


def kernel(pos_u, pos_v, neg_v, u_emb, W, b, v_emb):
    pu0 = pos_u[:, 0].astype(jnp.int32)
    pu1 = pos_u[:, 1].astype(jnp.int32)
    pv = pos_v.astype(jnp.int32)
    negf = neg_v.reshape(-1).astype(jnp.int32)

    w1, w2, vv, nv = _sc_gather(pu0, pu1, pv, negf, u_emb, v_emb)
    nv3 = nv.reshape(B, NNEG, PD)
    b2 = b.reshape(1, PD)

    out = _tc_loss(w1, w2, vv, nv3, W, b2)
    return -out[0, 0]
